# Initial kernel scaffold; baseline (speedup 1.0000x reference)
import jax
import jax.numpy as jnp
from jax.experimental import pallas as pl


def kernel(x, edge_index, batch, W1, b1, W2, b2, Wlin, blin):
    raise NotImplementedError("write your pallas kernel here")


# speedup vs baseline: 79.4140x; 79.4140x over previous
import functools

import jax
import jax.numpy as jnp
from jax import lax
from jax.experimental import pallas as pl
from jax.experimental.pallas import tpu as pltpu
from jax.experimental.pallas import tpu_sc as plsc

N = 100000
NG = 128
NC, NS, L = 2, 16, 16
NT = NC * NS
NP = 102400
SLC = NP // NS
SLA = NP // NT
E = 6400000
EP = 6422528
EPC = EP // NC
EPT = EP // NT
ECH = 2048
EROWS = EP // 128
NGP = 136

_mesh = plsc.VectorSubcoreMesh(
    core_axis_name="c", subcore_axis_name="s", num_cores=NC, num_subcores=NS)
_cparams = pltpu.CompilerParams(
    needs_layout_passes=False, use_tc_tiling_on_sc=False)


def _rsqrt16(d):
    i = plsc.bitcast(d, jnp.int32)
    i = 0x5F3759DF - lax.shift_right_arithmetic(i, 1)
    y = plsc.bitcast(i, jnp.float32)
    for _ in range(3):
        y = y * (1.5 - 0.5 * d * y * y)
    return y


def _iota16():
    return lax.iota(jnp.int32, 16)


@functools.partial(
    pl.kernel,
    out_type=(jax.ShapeDtypeStruct((NC, NP), jnp.float32),
              jax.ShapeDtypeStruct((NC, NGP), jnp.float32)),
    mesh=_mesh,
    compiler_params=_cparams,
    scratch_types=[
        pltpu.VMEM_SHARED((NP,), jnp.float32),
        pltpu.VMEM_SHARED((NGP,), jnp.float32),
        pltpu.VMEM((16, 128), jnp.int32),
        pltpu.VMEM((32, 128), jnp.int32),
        pltpu.VMEM((128,), jnp.float32),
        pltpu.VMEM((SLC,), jnp.float32),
    ],
)
def _k_deg(dst_hbm, batch_hbm, ones_hbm, z1_hbm, degp, cntp,
           deg_sh, cnt_sh, didx, bidx, ones_v, zb):
    c = lax.axis_index("c")
    s = lax.axis_index("s")
    pltpu.sync_copy(ones_hbm, ones_v)
    pltpu.sync_copy(z1_hbm, zb)
    pltpu.sync_copy(zb, deg_sh.at[pl.ds(s * SLC, SLC)])

    @pl.when(s == 0)
    def _():
        pltpu.sync_copy(zb.at[pl.ds(0, NGP)], cnt_sh)

    plsc.subcore_barrier()

    base_row = c * (EPC // 128) + s * (EPT // 128)

    @pl.loop(0, EPT // ECH)
    def _(g):
        pltpu.sync_copy(dst_hbm.at[pl.ds(base_row + g * 16, 16)], didx)
        for j in range(16):
            pltpu.sync_copy(ones_v, deg_sh.at[didx.at[j]], add=True)

    wid = s * NC + c
    row = wid * 25
    arow = (row // 8) * 8
    delta = row - arow
    pltpu.sync_copy(batch_hbm.at[pl.ds(arow, 32)], bidx)
    for j in range(25):
        pltpu.sync_copy(ones_v, cnt_sh.at[bidx.at[delta + j]], add=True)

    plsc.subcore_barrier()
    pltpu.sync_copy(deg_sh.at[pl.ds(s * SLC, SLC)],
                    degp.at[c].at[pl.ds(s * SLC, SLC)])

    @pl.when(s == 0)
    def _():
        pltpu.sync_copy(cnt_sh, cntp.at[c])


def _make_agg():
    @functools.partial(
        pl.kernel,
        out_type=jax.ShapeDtypeStruct((NC, NP, 8), jnp.float32),
        mesh=_mesh,
        compiler_params=_cparams,
        scratch_types=[
            pltpu.VMEM_SHARED((NP, 8), jnp.float32),
            pltpu.VMEM_SHARED((NP, 8), jnp.float32),
            pltpu.VMEM((16, 128), jnp.int32),
            pltpu.VMEM((16, 128), jnp.int32),
            pltpu.VMEM((128, 8), jnp.float32),
        ],
    )
    def _agg(t_hbm, src_hbm, dst_hbm, zf_hbm, outp,
             t_sh, z_sh, sidx, didx, vbuf):
        c = lax.axis_index("c")
        s = lax.axis_index("s")
        nb = s * SLC
        pltpu.sync_copy(t_hbm.at[pl.ds(nb, SLC)], t_sh.at[pl.ds(nb, SLC)])
        pltpu.sync_copy(zf_hbm, z_sh.at[pl.ds(nb, SLC)])
        plsc.subcore_barrier()

        base_row = c * (EPC // 128) + s * (EPT // 128)

        @pl.loop(0, EPT // ECH)
        def _(g):
            row0 = base_row + g * 16
            pltpu.sync_copy(src_hbm.at[pl.ds(row0, 16)], sidx)
            pltpu.sync_copy(dst_hbm.at[pl.ds(row0, 16)], didx)
            for j in range(16):
                pltpu.sync_copy(t_sh.at[sidx.at[j]], vbuf)
                pltpu.sync_copy(vbuf, z_sh.at[didx.at[j]], add=True)

        plsc.subcore_barrier()
        pltpu.sync_copy(z_sh.at[pl.ds(nb, SLC)],
                        outp.at[c].at[pl.ds(nb, SLC)])

    return _agg


_k_agg = _make_agg()


@functools.partial(
    pl.kernel,
    out_type=jax.ShapeDtypeStruct((NP, 2), jnp.float32),
    mesh=_mesh,
    compiler_params=_cparams,
    scratch_types=[
        pltpu.VMEM((SLA, 8), jnp.float32),
        pltpu.VMEM((SLA,), jnp.float32),
        pltpu.VMEM((64,), jnp.float32),
        pltpu.VMEM((16,), jnp.float32),
        pltpu.VMEM((32,), jnp.float32),
        pltpu.VMEM((SLA, 2), jnp.float32),
    ],
)
def _k_mlp(t1_hbm, dinv_hbm, w1_hbm, b1_hbm, w2_hbm, ut_out,
           tv, dv, w1v, b1v, w2v, utv):
    c = lax.axis_index("c")
    s = lax.axis_index("s")
    wid = s * NC + c
    nb = wid * SLA
    pltpu.sync_copy(t1_hbm.at[pl.ds(nb, SLA)], tv)
    pltpu.sync_copy(dinv_hbm.at[pl.ds(nb, SLA)], dv)
    pltpu.sync_copy(w1_hbm, w1v)
    pltpu.sync_copy(b1_hbm, b1v)
    pltpu.sync_copy(w2_hbm, w2v)
    w1rows = [w1v[pl.ds(k * 16, 16)] for k in range(4)]
    w1 = [[w1rows[k][j] for j in range(16)] for k in range(4)]
    b1row = b1v[pl.ds(0, 16)]
    b1 = [b1row[j] for j in range(16)]
    w2rows = [w2v[pl.ds(k * 16, 16)] for k in range(2)]
    w2 = [[w2rows[(2 * j + k) // 16][(2 * j + k) % 16]
           for k in range(2)] for j in range(16)]
    iota = _iota16()

    @pl.loop(0, SLA // 16)
    def _(i):
        dinv = dv[pl.ds(i * 16, 16)]
        ni = i * 16 + iota
        z = []
        for k in range(4):
            a = plsc.load_gather(tv, [ni, jnp.full((16,), k, jnp.int32)])
            b = plsc.load_gather(tv, [ni, jnp.full((16,), k + 4, jnp.int32)])
            z.append(dinv * (a + b))
        h1 = []
        for j in range(16):
            t = z[0] * w1[0][j] + z[1] * w1[1][j] + z[2] * w1[2][j] \
                + z[3] * w1[3][j] + b1[j]
            h1.append(jnp.maximum(t, 0.0))
        for k in range(2):
            u = h1[0] * w2[0][k]
            for j in range(1, 16):
                u = u + h1[j] * w2[j][k]
            kk = jnp.full((16,), k, jnp.int32)
            plsc.store_scatter(utv, [ni, kk], dinv * u)

    pltpu.sync_copy(utv, ut_out.at[pl.ds(nb, SLA)])


@functools.partial(
    pl.kernel,
    out_type=jax.ShapeDtypeStruct((NC, 2 * NGP), jnp.float32),
    mesh=_mesh,
    compiler_params=_cparams,
    scratch_types=[
        pltpu.VMEM_SHARED((2 * NGP,), jnp.float32),
        pltpu.VMEM((SLA, 4), jnp.float32),
        pltpu.VMEM((SLA,), jnp.float32),
        pltpu.VMEM((16,), jnp.float32),
        pltpu.VMEM((32, 128), jnp.int32),
        pltpu.VMEM((32, 128), jnp.int32),
        pltpu.VMEM((128,), jnp.float32),
        pltpu.VMEM((128,), jnp.float32),
        pltpu.VMEM((2 * NGP,), jnp.float32),
    ],
)
def _k_head(t2_hbm, dinv_hbm, coef_hbm, b0_hbm, b1_hbm, zp_hbm, poolp,
            pool_sh, tv, dv, coefv, bidx0, bidx1, h3c0, h3c1, pz):
    c = lax.axis_index("c")
    s = lax.axis_index("s")
    wid = s * NC + c
    nb = wid * SLA

    @pl.when(s == 0)
    def _():
        pltpu.sync_copy(zp_hbm, pool_sh)

    pltpu.sync_copy(t2_hbm.at[pl.ds(nb, SLA)], tv)
    pltpu.sync_copy(dinv_hbm.at[pl.ds(nb, SLA)], dv)
    pltpu.sync_copy(coef_hbm, coefv)
    brow = wid * 25
    barow = (brow // 8) * 8
    bdelta = brow - barow
    pltpu.sync_copy(b0_hbm.at[pl.ds(barow, 32)], bidx0)
    pltpu.sync_copy(b1_hbm.at[pl.ds(barow, 32)], bidx1)
    cv = coefv[pl.ds(0, 16)]
    b2 = [cv[0], cv[1]]
    wl = [[cv[2], cv[3]], [cv[4], cv[5]]]
    bl = [cv[6], cv[7]]
    iota = _iota16()
    plsc.subcore_barrier()

    @pl.loop(0, SLA // 128)
    def _(m):
        for i in range(8):
            nloc = m * 128 + i * 16
            dinv = dv[pl.ds(nloc, 16)]
            ni = nloc + iota
            h2 = []
            for k in range(2):
                a = plsc.load_gather(tv, [ni, jnp.full((16,), k, jnp.int32)])
                b = plsc.load_gather(
                    tv, [ni, jnp.full((16,), k + 2, jnp.int32)])
                h2.append(dinv * (a + b) + b2[k])
            li = i * 16 + iota
            h30 = h2[0] * wl[0][0] + h2[1] * wl[0][1] + bl[0]
            h31 = h2[0] * wl[1][0] + h2[1] * wl[1][1] + bl[1]
            plsc.store_scatter(h3c0, [li], h30)
            plsc.store_scatter(h3c1, [li], h31)
        pltpu.sync_copy(h3c0, pool_sh.at[bidx0.at[bdelta + m]], add=True)
        pltpu.sync_copy(h3c1, pool_sh.at[bidx1.at[bdelta + m]], add=True)

    plsc.subcore_barrier()

    @pl.when(s == 0)
    def _():
        pltpu.sync_copy(pool_sh, pz)
        pltpu.sync_copy(pz, poolp.at[c])


def kernel(x, edge_index, batch, W1, b1, W2, b2, Wlin, blin):
    src = edge_index[0].astype(jnp.int32)
    dst = edge_index[1].astype(jnp.int32)
    epad = jnp.full((EP - E,), N, jnp.int32)
    src2d = jnp.concatenate([src, epad]).reshape(EROWS, 128)
    dst2d = jnp.concatenate([dst, epad]).reshape(EROWS, 128)
    batch2d = jnp.concatenate(
        [batch.astype(jnp.int32), jnp.full((NP - N,), NG, jnp.int32)]
    ).reshape(NP // 128, 128)
    xp = jnp.pad(x, ((0, NP - N), (0, 0)))
    ones128 = jnp.ones((128,), jnp.float32)
    z1 = jnp.zeros((SLC,), jnp.float32)
    z8 = jnp.zeros((SLC, 8), jnp.float32)
    zp = jnp.zeros((2 * NGP,), jnp.float32)
    b2d0 = batch2d * 2
    b2d1 = batch2d * 2 + 1
    coef = jnp.concatenate(
        [b2, Wlin.reshape(-1), blin, jnp.zeros((8,), jnp.float32)])

    degp, cntp = _k_deg(dst2d, batch2d, ones128, z1)
    dinv = lax.rsqrt(degp[0] + degp[1] + 1.0)
    xt = xp * dinv[:, None]
    xt8 = jnp.pad(xt, ((0, 0), (0, 4)))
    s1p = _k_agg(xt8, src2d, dst2d, z8)
    t1 = jnp.concatenate([s1p[0, :, :4] + s1p[1, :, :4], xt], axis=1)
    ut = _k_mlp(t1, dinv, W1.reshape(-1), b1, W2.reshape(-1))
    ut8 = jnp.pad(ut, ((0, 0), (0, 6)))
    s2p = _k_agg(ut8, src2d, dst2d, z8)
    t2 = jnp.concatenate([s2p[0, :, :2] + s2p[1, :, :2], ut], axis=1)
    poolp = _k_head(t2, dinv, coef, b2d0, b2d1, zp)

    pool = (poolp[0] + poolp[1])[:2 * NG].reshape(NG, 2)
    cnt = cntp[0, :NG] + cntp[1, :NG]
    return pool / jnp.maximum(cnt, 1.0)[:, None]
